# baseline (speedup 1.0000x reference)
import functools

import jax
import jax.numpy as jnp
from jax import lax
from jax.experimental import pallas as pl
from jax.experimental.pallas import tpu as pltpu
from jax.experimental.pallas import tpu_sc as plsc


_BS = 1024


def _add_body(x_ref, pe_ref, o_ref):
    o_ref[...] = x_ref[...] + pe_ref[...][None, :, :]


def _tc_kernel(x, pe_weight):
    B, S, D = x.shape
    grid = (S // _BS, 2)
    return pl.pallas_call(
        _add_body,
        grid=grid,
        in_specs=[
            pl.BlockSpec((2, _BS, D), lambda s, b: (b, s, 0)),
            pl.BlockSpec((_BS, D), lambda s, b: (s, 0)),
        ],
        out_specs=pl.BlockSpec((2, _BS, D), lambda s, b: (b, s, 0)),
        out_shape=jax.ShapeDtypeStruct((B, S, D), x.dtype),
    )(x, pe_weight)


_NC = 2
_NS = 16
_NW = _NC * _NS
_LANES = 16
_CH = 32
_UNROLL = 8


def _sc_add_body(x_hbm, pe_hbm, out_hbm,
                 xb0, xb1, pb, si0, si1, sp, so0, so1):
    D = 1024
    rows_w = 1024
    chw = _CH * D
    nch = rows_w // _CH
    wpb = 8

    wid = lax.axis_index("s") * _NC + lax.axis_index("c")
    xbase = wid * (rows_w * D)
    pbase = (wid % wpb) * (rows_w * D)

    bufs = ((xb0, si0, so0), (xb1, si1, so1))

    def start_x_in(c):
        xb, si, _ = bufs[c % 2]
        return pltpu.async_copy(x_hbm.at[pl.ds(xbase + c * chw, chw)], xb, si)

    def start_pe_in(c):
        return pltpu.async_copy(pe_hbm.at[pl.ds(pbase + c * chw, chw)], pb, sp)

    def inner_add(xb, pb):
        @plsc.parallel_loop(0, chw // _LANES, 1, unroll=_UNROLL)
        def _body(i):
            off = i * _LANES
            xb[pl.ds(off, _LANES)] = (
                xb[pl.ds(off, _LANES)] + pb[pl.ds(off, _LANES)]
            )

    in_h = [None, None]
    out_h = [None, None]
    in_h[0] = start_x_in(0)
    pe_h = start_pe_in(0)
    for c in range(nch):
        b = c % 2
        if c + 1 < nch:
            if out_h[1 - b] is not None:
                out_h[1 - b].wait()
            in_h[1 - b] = start_x_in(c + 1)
        in_h[b].wait()
        pe_h.wait()
        xb, _, so = bufs[b]
        inner_add(xb, pb)
        if c + 1 < nch:
            pe_h = start_pe_in(c + 1)
        out_h[b] = pltpu.async_copy(
            xb, out_hbm.at[pl.ds(xbase + c * chw, chw)], so)
    for h in out_h:
        if h is not None:
            h.wait()


def _sc_kernel(x, pe_weight):
    B, S, D = x.shape
    n = B * S * D
    sc_add = functools.partial(
        pl.kernel,
        mesh=plsc.VectorSubcoreMesh(core_axis_name="c", subcore_axis_name="s"),
        out_type=jax.ShapeDtypeStruct((n,), jnp.float32),
        scratch_types=[
            pltpu.VMEM((_CH * D,), jnp.float32),
            pltpu.VMEM((_CH * D,), jnp.float32),
            pltpu.VMEM((_CH * D,), jnp.float32),
            pltpu.SemaphoreType.DMA,
            pltpu.SemaphoreType.DMA,
            pltpu.SemaphoreType.DMA,
            pltpu.SemaphoreType.DMA,
            pltpu.SemaphoreType.DMA,
        ],
    )(_sc_add_body)
    out = sc_add(x.reshape(n), pe_weight.reshape(S * D))
    return out.reshape(B, S, D)


def kernel(x, pe_weight):
    B, S, D = x.shape
    max_len = pe_weight.shape[0]
    assert S == max_len
    return _sc_kernel(x, pe_weight)


# speedup vs baseline: 2.3943x; 2.3943x over previous
import functools

import jax
import jax.numpy as jnp
from jax import lax
from jax.experimental import pallas as pl
from jax.experimental.pallas import tpu as pltpu
from jax.experimental.pallas import tpu_sc as plsc


_BS = 1024


def _add_body(x_ref, pe_ref, o_ref):
    o_ref[...] = x_ref[...] + pe_ref[...][None, :, :]


def _tc_kernel(x, pe_weight):
    B, S, D = x.shape
    grid = (S // _BS, 2)
    return pl.pallas_call(
        _add_body,
        grid=grid,
        in_specs=[
            pl.BlockSpec((2, _BS, D), lambda s, b: (b, s, 0)),
            pl.BlockSpec((_BS, D), lambda s, b: (s, 0)),
        ],
        out_specs=pl.BlockSpec((2, _BS, D), lambda s, b: (b, s, 0)),
        out_shape=jax.ShapeDtypeStruct((B, S, D), x.dtype),
    )(x, pe_weight)


_NC = 2
_NS = 16
_NW = _NC * _NS
_LANES = 16
_CH = 32
_UNROLL = 8


def _sc_add_body(x_hbm, pe_hbm, out_hbm,
                 xb0, xb1, pb, si0, si1, sp, so0, so1):
    D = 1024
    seq_w = 256
    rows_w = 1024
    nch = rows_w // _CH
    wpb = 8

    wid = lax.axis_index("s") * _NC + lax.axis_index("c")
    batch = wid // wpb
    sbase = (wid % wpb) * rows_w

    bufs = ((xb0, si0, so0), (xb1, si1, so1))

    def start_x_in(c):
        xb, si, _ = bufs[c % 2]
        return pltpu.async_copy(
            x_hbm.at[batch, pl.ds(sbase + c * _CH, _CH)], xb, si)

    def start_pe_in(c):
        return pltpu.async_copy(
            pe_hbm.at[pl.ds(sbase + c * _CH, _CH)], pb, sp)

    def inner_add(xb):
        @plsc.parallel_loop(0, _CH * (D // _LANES), 1, unroll=_UNROLL)
        def _body(i):
            r = lax.shift_right_logical(i, 6)
            col = pl.multiple_of(
                lax.shift_left(lax.bitwise_and(i, D // _LANES - 1), 4), _LANES)
            xb[r, pl.ds(col, _LANES)] = (
                xb[r, pl.ds(col, _LANES)] + pb[r, pl.ds(col, _LANES)]
            )

    in_h = [None, None]
    out_h = [None, None]
    in_h[0] = start_x_in(0)
    pe_h = start_pe_in(0)
    for c in range(nch):
        b = c % 2
        if c + 1 < nch:
            if out_h[1 - b] is not None:
                out_h[1 - b].wait()
            in_h[1 - b] = start_x_in(c + 1)
        in_h[b].wait()
        pe_h.wait()
        xb, _, so = bufs[b]
        inner_add(xb)
        if c + 1 < nch:
            pe_h = start_pe_in(c + 1)
        out_h[b] = pltpu.async_copy(
            xb, out_hbm.at[batch, pl.ds(sbase + c * _CH, _CH)], so)
    for h in out_h:
        if h is not None:
            h.wait()


def _sc_kernel(x, pe_weight):
    B, S, D = x.shape
    sc_add = functools.partial(
        pl.kernel,
        mesh=plsc.VectorSubcoreMesh(core_axis_name="c", subcore_axis_name="s"),
        out_type=jax.ShapeDtypeStruct((B, S, D), jnp.float32),
        scratch_types=[
            pltpu.VMEM((_CH, D), jnp.float32),
            pltpu.VMEM((_CH, D), jnp.float32),
            pltpu.VMEM((_CH, D), jnp.float32),
            pltpu.SemaphoreType.DMA,
            pltpu.SemaphoreType.DMA,
            pltpu.SemaphoreType.DMA,
            pltpu.SemaphoreType.DMA,
            pltpu.SemaphoreType.DMA,
        ],
    )(_sc_add_body)
    return sc_add(x, pe_weight)


def kernel(x, pe_weight):
    B, S, D = x.shape
    max_len = pe_weight.shape[0]
    assert S == max_len
    return _sc_kernel(x, pe_weight)


# speedup vs baseline: 2.5001x; 1.0442x over previous
import functools

import jax
import jax.numpy as jnp
from jax import lax
from jax.experimental import pallas as pl
from jax.experimental.pallas import tpu as pltpu
from jax.experimental.pallas import tpu_sc as plsc


_BS = 1024


def _add_body(x_ref, pe_ref, o_ref):
    o_ref[...] = x_ref[...] + pe_ref[...][None, :, :]


def _tc_kernel(x, pe_weight):
    B, S, D = x.shape
    grid = (S // _BS, 2)
    return pl.pallas_call(
        _add_body,
        grid=grid,
        in_specs=[
            pl.BlockSpec((2, _BS, D), lambda s, b: (b, s, 0)),
            pl.BlockSpec((_BS, D), lambda s, b: (s, 0)),
        ],
        out_specs=pl.BlockSpec((2, _BS, D), lambda s, b: (b, s, 0)),
        out_shape=jax.ShapeDtypeStruct((B, S, D), x.dtype),
    )(x, pe_weight)


_NC = 2
_NS = 16
_NW = _NC * _NS
_LANES = 16
_CH = 16
_UNROLL = 16


def _sc_add_body(x_hbm, pe_hbm, out_hbm,
                 xb0, pb0, xb1, pb1, si0, si1, so0, so1):
    D = 1024
    seq_w = 256
    rows_w = 1024
    nch = rows_w // _CH
    wpb = 8

    wid = lax.axis_index("s") * _NC + lax.axis_index("c")
    batch = wid // wpb
    sbase = (wid % wpb) * rows_w

    bufs = ((xb0, pb0, si0, so0), (xb1, pb1, si1, so1))

    def start_in(c):
        xb, pb, si, _ = bufs[c % 2]
        hx = pltpu.async_copy(
            x_hbm.at[batch, pl.ds(sbase + c * _CH, _CH)], xb, si)
        hp = pltpu.async_copy(
            pe_hbm.at[pl.ds(sbase + c * _CH, _CH)], pb, si)
        return (hx, hp)

    def inner_add(xb, pb):
        @plsc.parallel_loop(0, _CH * (D // _LANES), 1, unroll=_UNROLL)
        def _body(i):
            r = lax.shift_right_logical(i, 6)
            col = pl.multiple_of(
                lax.shift_left(lax.bitwise_and(i, D // _LANES - 1), 4), _LANES)
            xb[r, pl.ds(col, _LANES)] = (
                xb[r, pl.ds(col, _LANES)] + pb[r, pl.ds(col, _LANES)]
            )

    in_h = [None, None]
    out_h = [None, None]
    in_h[0] = start_in(0)
    for c in range(nch):
        b = c % 2
        if c + 1 < nch:
            if out_h[1 - b] is not None:
                out_h[1 - b].wait()
            in_h[1 - b] = start_in(c + 1)
        for h in in_h[b]:
            h.wait()
        xb, pb, _, so = bufs[b]
        inner_add(xb, pb)
        out_h[b] = pltpu.async_copy(
            xb, out_hbm.at[batch, pl.ds(sbase + c * _CH, _CH)], so)
    for h in out_h:
        if h is not None:
            h.wait()


def _sc_kernel(x, pe_weight):
    B, S, D = x.shape
    sc_add = functools.partial(
        pl.kernel,
        mesh=plsc.VectorSubcoreMesh(core_axis_name="c", subcore_axis_name="s"),
        out_type=jax.ShapeDtypeStruct((B, S, D), jnp.float32),
        scratch_types=[
            pltpu.VMEM((_CH, D), jnp.float32),
            pltpu.VMEM((_CH, D), jnp.float32),
            pltpu.VMEM((_CH, D), jnp.float32),
            pltpu.VMEM((_CH, D), jnp.float32),
            pltpu.SemaphoreType.DMA,
            pltpu.SemaphoreType.DMA,
            pltpu.SemaphoreType.DMA,
            pltpu.SemaphoreType.DMA,
        ],
    )(_sc_add_body)
    return sc_add(x, pe_weight)


def kernel(x, pe_weight):
    B, S, D = x.shape
    max_len = pe_weight.shape[0]
    assert S == max_len
    return _sc_kernel(x, pe_weight)


# speedup vs baseline: 4.7748x; 1.9099x over previous
import jax
from jax.experimental import pallas as pl


_BS = 1024
_BB = 2


def _add_body(x_ref, pe_ref, o_ref):
    o_ref[...] = x_ref[...] + pe_ref[...][None, :, :]


def kernel(x, pe_weight):
    B, S, D = x.shape
    max_len = pe_weight.shape[0]
    assert S == max_len

    grid = (S // _BS, B // _BB)
    return pl.pallas_call(
        _add_body,
        grid=grid,
        in_specs=[
            pl.BlockSpec((_BB, _BS, D), lambda s, b: (b, s, 0)),
            pl.BlockSpec((_BS, D), lambda s, b: (s, 0)),
        ],
        out_specs=pl.BlockSpec((_BB, _BS, D), lambda s, b: (b, s, 0)),
        out_shape=jax.ShapeDtypeStruct((B, S, D), x.dtype),
    )(x, pe_weight)
